# baseline (speedup 1.0000x reference)
import functools

import jax
import jax.numpy as jnp
from jax import lax
from jax.experimental import pallas as pl
from jax.experimental.pallas import tpu as pltpu
from jax.experimental.pallas import tpu_sc as plsc

B, L, H = 16, 512, 768
PAD = 1
NTOK = B * L
NC, NS = 2, 16
NW = NC * NS
CH = 64
POS_K = 520
SPAT_K = 1024
RPS = 2
M2 = RPS * L
NCHUNK = 2
BC = B // NCHUNK
GBC = BC // RPS
NTOKC = BC * L


def _sc_gather_words(word_emb, idx2d):
    chunks = NTOKC // (NW * CH)
    mesh = plsc.VectorSubcoreMesh(core_axis_name="c", subcore_axis_name="s")

    @functools.partial(
        pl.kernel,
        out_type=jax.ShapeDtypeStruct((NTOKC, H), jnp.float32),
        mesh=mesh,
        scratch_types=[
            pltpu.VMEM((chunks, CH), jnp.int32),
            pltpu.VMEM((CH, H), jnp.float32),
            pltpu.VMEM((CH, H), jnp.float32),
            pltpu.SemaphoreType.DMA,
            pltpu.SemaphoreType.DMA,
        ],
    )
    def gather_kernel(table_hbm, idx_hbm, out_hbm, idx_v, rows0, rows1, sem0, sem1):
        wid = lax.axis_index("s") * NC + lax.axis_index("c")
        row0 = wid * chunks
        pltpu.sync_copy(idx_hbm.at[pl.ds(row0, chunks)], idx_v)
        bufs = (rows0, rows1)
        sems = (sem0, sem1)

        def start(c):
            return pltpu.async_copy(table_hbm.at[idx_v.at[c]], bufs[c % 2],
                                    sems[c % 2])

        copies = [start(0)] + ([start(1)] if chunks > 1 else [])
        for c in range(chunks):
            copies[c % 2].wait()
            pltpu.sync_copy(bufs[c % 2], out_hbm.at[pl.ds((row0 + c) * CH, CH)])
            if c + 2 < chunks:
                copies[c % 2] = start(c + 2)

    return gather_kernel(word_emb, idx2d)


def _tc_body(w_ref, ids_ref, bb_ref, pos_ref, x_ref, y_ref, h_ref, ww_ref,
             g_ref, b_ref, o_ref):
    ids = ids_ref[0]
    mask = (ids != PAD).astype(jnp.int32)
    c = mask
    sh = 1
    while sh < M2:
        shifted = jnp.concatenate(
            [jnp.zeros((sh, 1), jnp.int32), c[: M2 - sh]], axis=0)
        c = c + shifted
        sh *= 2
    rowi = lax.broadcasted_iota(jnp.int32, (M2, 1), 0)
    carry = c[L - 1:L, :]
    c = c - jnp.where(rowi >= L, carry, 0)
    pids = c * mask + PAD

    def onehot(col_idx, k):
        io = lax.broadcasted_iota(jnp.int32, (M2, k), 1)
        return (io == col_idx).astype(jnp.bfloat16)

    def mm(oh, t_ref):
        return lax.dot_general(oh, t_ref[...], (((1,), (0,)), ((), ())),
                               preferred_element_type=jnp.float32)

    bb = bb_ref[0]
    b0 = bb[:, 0:1]
    b1 = bb[:, 1:2]
    b2 = bb[:, 2:3]
    b3 = bb[:, 3:4]
    hi = jnp.clip(b3 - b1, 0, SPAT_K - 1)
    wi = jnp.clip(b2 - b0, 0, SPAT_K - 1)

    pos_part = mm(onehot(pids, POS_K), pos_ref)
    left = mm(onehot(b0, SPAT_K), x_ref)
    upper = mm(onehot(b1, SPAT_K), y_ref)
    right = mm(onehot(b2, SPAT_K), x_ref)
    lower = mm(onehot(b3, SPAT_K), y_ref)
    hgt = mm(onehot(hi, SPAT_K), h_ref)
    wid = mm(onehot(wi, SPAT_K), ww_ref)
    spatial = jnp.concatenate([left, upper, right, lower, hgt, wid], axis=-1)

    acc = w_ref[0] + pos_part + spatial
    mu = jnp.mean(acc, axis=-1, keepdims=True)
    d = acc - mu
    var = jnp.mean(d * d, axis=-1, keepdims=True)
    o_ref[0] = d * lax.rsqrt(var + 1e-5) * g_ref[...] + b_ref[...]


def _tc_fuse(w_rows, ids3, bbox3, pos_t, x_t, y_t, h_t, w_t, g_row, b_row):
    return pl.pallas_call(
        _tc_body,
        grid=(GBC,),
        compiler_params=pltpu.CompilerParams(
            dimension_semantics=("parallel",)),
        in_specs=[
            pl.BlockSpec((1, M2, H), lambda i: (i, 0, 0)),
            pl.BlockSpec((1, M2, 1), lambda i: (i, 0, 0)),
            pl.BlockSpec((1, M2, 4), lambda i: (i, 0, 0)),
            pl.BlockSpec((POS_K, H), lambda i: (0, 0)),
            pl.BlockSpec((SPAT_K, 128), lambda i: (0, 0)),
            pl.BlockSpec((SPAT_K, 128), lambda i: (0, 0)),
            pl.BlockSpec((SPAT_K, 128), lambda i: (0, 0)),
            pl.BlockSpec((SPAT_K, 128), lambda i: (0, 0)),
            pl.BlockSpec((1, H), lambda i: (0, 0)),
            pl.BlockSpec((1, H), lambda i: (0, 0)),
        ],
        out_specs=pl.BlockSpec((1, M2, H), lambda i: (i, 0, 0)),
        out_shape=jax.ShapeDtypeStruct((GBC, M2, H), jnp.float32),
    )(w_rows, ids3, bbox3, pos_t, x_t, y_t, h_t, w_t, g_row, b_row)


def kernel(input_ids, bbox, word_emb, token_type_emb, pos_emb, x_emb, y_emb,
           h_emb, w_emb, ln_g, ln_b):
    pos_t = jnp.zeros((POS_K, H), jnp.bfloat16).at[:514].set(
        (pos_emb + token_type_emb[0:1]).astype(jnp.bfloat16))
    x_t = x_emb.astype(jnp.bfloat16)
    y_t = y_emb.astype(jnp.bfloat16)
    h_t = h_emb.astype(jnp.bfloat16)
    w_t = w_emb.astype(jnp.bfloat16)
    g_row = ln_g.reshape(1, H)
    b_row = ln_b.reshape(1, H)

    ids_c = input_ids.reshape(NCHUNK, NTOKC // CH, CH)
    ids3_c = input_ids.reshape(NCHUNK, GBC, M2, 1)
    bbox3_c = bbox.reshape(NCHUNK, GBC, M2, 4)

    w_chunks = [_sc_gather_words(word_emb, ids_c[k]) for k in range(NCHUNK)]
    outs = [
        _tc_fuse(w_chunks[k].reshape(GBC, M2, H), ids3_c[k], bbox3_c[k],
                 pos_t, x_t, y_t, h_t, w_t, g_row, b_row)
        for k in range(NCHUNK)
    ]
    return jnp.concatenate(outs, axis=0).reshape(B, L, H)


# speedup vs baseline: 1.3290x; 1.3290x over previous
import functools

import jax
import jax.numpy as jnp
from jax import lax
from jax.experimental import pallas as pl
from jax.experimental.pallas import tpu as pltpu
from jax.experimental.pallas import tpu_sc as plsc

B, L, H = 16, 512, 768
PAD = 1
NTOK = B * L
NC, NS = 2, 16
NW = NC * NS
CH = 64
ROWS_PER_TILE = NTOK // NW
CHUNKS = ROWS_PER_TILE // CH
POS_K = 520
SPAT_K = 1024


def _sc_gather_words(word_emb, idx_flat):
    mesh = plsc.VectorSubcoreMesh(core_axis_name="c", subcore_axis_name="s")

    @functools.partial(
        pl.kernel,
        out_type=jax.ShapeDtypeStruct((NTOK, H), jnp.float32),
        mesh=mesh,
        scratch_types=[
            pltpu.VMEM((ROWS_PER_TILE,), jnp.int32),
            pltpu.VMEM((CH, H), jnp.float32),
            pltpu.VMEM((CH, H), jnp.float32),
            pltpu.SemaphoreType.DMA,
            pltpu.SemaphoreType.DMA,
        ],
    )
    def gather_kernel(table_hbm, idx_hbm, out_hbm, idx_v, rows0, rows1, sem0, sem1):
        wid = lax.axis_index("s") * NC + lax.axis_index("c")
        base = wid * ROWS_PER_TILE
        pltpu.sync_copy(idx_hbm.at[pl.ds(base, ROWS_PER_TILE)], idx_v)
        bufs = (rows0, rows1)
        sems = (sem0, sem1)

        def start(c):
            return pltpu.async_copy(
                table_hbm.at[idx_v.at[pl.ds(c * CH, CH)]], bufs[c % 2],
                sems[c % 2])

        copies = [start(0), start(1)]
        for c in range(CHUNKS):
            copies[c % 2].wait()
            pltpu.sync_copy(bufs[c % 2], out_hbm.at[pl.ds(base + c * CH, CH)])
            if c + 2 < CHUNKS:
                copies[c % 2] = start(c + 2)

    return gather_kernel(word_emb, idx_flat)


def _tc_body(w_ref, idx_ref, pos_ref, x_ref, y_ref, h_ref, ww_ref,
             g_ref, b_ref, o_ref):
    idx = idx_ref[0]

    def onehot_t(row, k):
        io = lax.broadcasted_iota(jnp.int32, (k, L), 0)
        return (io == idx[row:row + 1, :]).astype(jnp.bfloat16)

    def mm_t(oh_t, table_bf):
        return lax.dot_general(oh_t, table_bf, (((0,), (0,)), ((), ())),
                               preferred_element_type=jnp.float32)

    pos_part = mm_t(onehot_t(0, POS_K), pos_ref[...].astype(jnp.bfloat16))
    x_bf = x_ref[...].astype(jnp.bfloat16)
    y_bf = y_ref[...].astype(jnp.bfloat16)
    left = mm_t(onehot_t(1, SPAT_K), x_bf)
    upper = mm_t(onehot_t(2, SPAT_K), y_bf)
    right = mm_t(onehot_t(3, SPAT_K), x_bf)
    lower = mm_t(onehot_t(4, SPAT_K), y_bf)
    hgt = mm_t(onehot_t(5, SPAT_K), h_ref[...].astype(jnp.bfloat16))
    wid = mm_t(onehot_t(6, SPAT_K), ww_ref[...].astype(jnp.bfloat16))
    spatial = jnp.concatenate([left, upper, right, lower, hgt, wid], axis=-1)

    acc = w_ref[0] + pos_part + spatial
    mu = jnp.mean(acc, axis=-1, keepdims=True)
    d = acc - mu
    var = jnp.mean(d * d, axis=-1, keepdims=True)
    o_ref[0] = d * lax.rsqrt(var + 1e-5) * g_ref[...] + b_ref[...]


def _tc_fuse(w_rows, idx_t, pos_t, x_emb, y_emb, h_emb, w_emb, g_row, b_row):
    return pl.pallas_call(
        _tc_body,
        grid=(B,),
        compiler_params=pltpu.CompilerParams(
            dimension_semantics=("parallel",)),
        in_specs=[
            pl.BlockSpec((1, L, H), lambda i: (i, 0, 0)),
            pl.BlockSpec((1, 8, L), lambda i: (i, 0, 0)),
            pl.BlockSpec((POS_K, H), lambda i: (0, 0)),
            pl.BlockSpec((SPAT_K, 128), lambda i: (0, 0)),
            pl.BlockSpec((SPAT_K, 128), lambda i: (0, 0)),
            pl.BlockSpec((SPAT_K, 128), lambda i: (0, 0)),
            pl.BlockSpec((SPAT_K, 128), lambda i: (0, 0)),
            pl.BlockSpec((1, H), lambda i: (0, 0)),
            pl.BlockSpec((1, H), lambda i: (0, 0)),
        ],
        out_specs=pl.BlockSpec((1, L, H), lambda i: (i, 0, 0)),
        out_shape=jax.ShapeDtypeStruct((B, L, H), jnp.float32),
    )(w_rows, idx_t, pos_t, x_emb, y_emb, h_emb, w_emb, g_row, b_row)


def kernel(input_ids, bbox, word_emb, token_type_emb, pos_emb, x_emb, y_emb,
           h_emb, w_emb, ln_g, ln_b):
    idx_flat = input_ids.reshape(NTOK)
    w_rows = _sc_gather_words(word_emb, idx_flat).reshape(B, L, H)

    mask = (input_ids != PAD).astype(jnp.int32)
    pids = jnp.cumsum(mask, axis=1) * mask + PAD
    b0 = bbox[:, :, 0]
    b1 = bbox[:, :, 1]
    b2 = bbox[:, :, 2]
    b3 = bbox[:, :, 3]
    hi = jnp.clip(b3 - b1, 0, SPAT_K - 1)
    wi = jnp.clip(b2 - b0, 0, SPAT_K - 1)
    idx_t = jnp.stack([pids, b0, b1, b2, b3, hi, wi, pids], axis=1)

    pos_t = jnp.zeros((POS_K, H), jnp.float32).at[:514].set(
        pos_emb + token_type_emb[0:1])
    g_row = ln_g.reshape(1, H)
    b_row = ln_b.reshape(1, H)
    return _tc_fuse(w_rows, idx_t, pos_t, x_emb, y_emb, h_emb, w_emb,
                    g_row, b_row)


# speedup vs baseline: 1.3463x; 1.0130x over previous
import functools

import jax
import jax.numpy as jnp
from jax import lax
from jax.experimental import pallas as pl
from jax.experimental.pallas import tpu as pltpu
from jax.experimental.pallas import tpu_sc as plsc

B, L, H = 16, 512, 768
PAD = 1
NTOK = B * L
NC, NS = 2, 16
NW = NC * NS
CH = 64
POS_K = 520
SPAT_K = 1024
NCHUNK = 2
BC = B // NCHUNK
NTOKC = BC * L
ROWS_PER_TILE = NTOKC // NW
CHUNKS = ROWS_PER_TILE // CH


def _sc_gather_words(word_emb, idx_flat):
    mesh = plsc.VectorSubcoreMesh(core_axis_name="c", subcore_axis_name="s")

    @functools.partial(
        pl.kernel,
        out_type=jax.ShapeDtypeStruct((NTOKC, H), jnp.float32),
        mesh=mesh,
        scratch_types=[
            pltpu.VMEM((ROWS_PER_TILE,), jnp.int32),
            pltpu.VMEM((CH, H), jnp.float32),
            pltpu.VMEM((CH, H), jnp.float32),
            pltpu.SemaphoreType.DMA,
            pltpu.SemaphoreType.DMA,
        ],
    )
    def gather_kernel(table_hbm, idx_hbm, out_hbm, idx_v, rows0, rows1, sem0, sem1):
        wid = lax.axis_index("s") * NC + lax.axis_index("c")
        base = wid * ROWS_PER_TILE
        pltpu.sync_copy(idx_hbm.at[pl.ds(base, ROWS_PER_TILE)], idx_v)
        bufs = (rows0, rows1)
        sems = (sem0, sem1)

        def start(c):
            return pltpu.async_copy(
                table_hbm.at[idx_v.at[pl.ds(c * CH, CH)]], bufs[c % 2],
                sems[c % 2])

        copies = [start(0)] + ([start(1)] if CHUNKS > 1 else [])
        for c in range(CHUNKS):
            copies[c % 2].wait()
            pltpu.sync_copy(bufs[c % 2], out_hbm.at[pl.ds(base + c * CH, CH)])
            if c + 2 < CHUNKS:
                copies[c % 2] = start(c + 2)

    return gather_kernel(word_emb, idx_flat)


def _tc_body(w_ref, idx_ref, io_ref, pos_ref, x_ref, y_ref, h_ref, ww_ref,
             g_ref, b_ref, *prev_and_out):
    o_ref = prev_and_out[-1]
    idx = idx_ref[0]
    io_spat = io_ref[...]

    def onehot_t(row, k):
        io = io_spat[:POS_K] if k == POS_K else io_spat
        return (io == idx[row:row + 1, :]).astype(jnp.bfloat16)

    def mm_t(oh_t, table_bf):
        return lax.dot_general(oh_t, table_bf, (((0,), (0,)), ((), ())),
                               preferred_element_type=jnp.float32)

    pos_part = mm_t(onehot_t(0, POS_K), pos_ref[...].astype(jnp.bfloat16))
    x_bf = x_ref[...].astype(jnp.bfloat16)
    y_bf = y_ref[...].astype(jnp.bfloat16)
    left = mm_t(onehot_t(1, SPAT_K), x_bf)
    upper = mm_t(onehot_t(2, SPAT_K), y_bf)
    right = mm_t(onehot_t(3, SPAT_K), x_bf)
    lower = mm_t(onehot_t(4, SPAT_K), y_bf)
    hgt = mm_t(onehot_t(5, SPAT_K), h_ref[...].astype(jnp.bfloat16))
    wid = mm_t(onehot_t(6, SPAT_K), ww_ref[...].astype(jnp.bfloat16))
    spatial = jnp.concatenate([left, upper, right, lower, hgt, wid], axis=-1)

    acc = w_ref[0] + pos_part + spatial
    mu = jnp.mean(acc, axis=-1, keepdims=True)
    d = acc - mu
    var = jnp.mean(d * d, axis=-1, keepdims=True)
    o_ref[0] = d * lax.rsqrt(var + 1e-5) * g_ref[...] + b_ref[...]


def _tc_fuse(chunk, w_rows, idx_t, io_arr, pos_t, x_emb, y_emb, h_emb, w_emb,
             g_row, b_row, prev):
    base = chunk * BC
    specs = [
        pl.BlockSpec((1, L, H), lambda i: (i, 0, 0)),
        pl.BlockSpec((1, 8, L), lambda i: (i, 0, 0)),
        pl.BlockSpec((SPAT_K, L), lambda i: (0, 0)),
        pl.BlockSpec((POS_K, H), lambda i: (0, 0)),
        pl.BlockSpec((SPAT_K, 128), lambda i: (0, 0)),
        pl.BlockSpec((SPAT_K, 128), lambda i: (0, 0)),
        pl.BlockSpec((SPAT_K, 128), lambda i: (0, 0)),
        pl.BlockSpec((SPAT_K, 128), lambda i: (0, 0)),
        pl.BlockSpec((1, H), lambda i: (0, 0)),
        pl.BlockSpec((1, H), lambda i: (0, 0)),
    ]
    args = [w_rows, idx_t, io_arr, pos_t, x_emb, y_emb, h_emb, w_emb,
            g_row, b_row]
    aliases = {}
    if prev is not None:
        specs.append(pl.BlockSpec(memory_space=pl.ANY))
        args.append(prev)
        aliases = {10: 0}
    return pl.pallas_call(
        _tc_body,
        grid=(BC,),
        compiler_params=pltpu.CompilerParams(
            dimension_semantics=("arbitrary",)),
        in_specs=specs,
        out_specs=pl.BlockSpec((1, L, H), lambda i, b=base: (b + i, 0, 0)),
        out_shape=jax.ShapeDtypeStruct((B, L, H), jnp.float32),
        input_output_aliases=aliases,
    )(*args)


def kernel(input_ids, bbox, word_emb, token_type_emb, pos_emb, x_emb, y_emb,
           h_emb, w_emb, ln_g, ln_b):
    mask = (input_ids != PAD).astype(jnp.int32)
    pids = jnp.cumsum(mask, axis=1) * mask + PAD
    b0 = bbox[:, :, 0]
    b1 = bbox[:, :, 1]
    b2 = bbox[:, :, 2]
    b3 = bbox[:, :, 3]
    hi = jnp.clip(b3 - b1, 0, SPAT_K - 1)
    wi = jnp.clip(b2 - b0, 0, SPAT_K - 1)
    idx_t = jnp.stack([pids, b0, b1, b2, b3, hi, wi, pids], axis=1)

    io_arr = lax.broadcasted_iota(jnp.int32, (SPAT_K, L), 0)
    pos_t = jnp.zeros((POS_K, H), jnp.float32).at[:514].set(
        pos_emb + token_type_emb[0:1])
    g_row = ln_g.reshape(1, H)
    b_row = ln_b.reshape(1, H)

    idx_flat = input_ids.reshape(NCHUNK, NTOKC)
    idx_t_c = idx_t.reshape(NCHUNK, BC, 8, L)

    w_chunks = [_sc_gather_words(word_emb, idx_flat[k]).reshape(BC, L, H)
                for k in range(NCHUNK)]
    prev = None
    for k in range(NCHUNK):
        prev = _tc_fuse(k, w_chunks[k], idx_t_c[k], io_arr, pos_t,
                        x_emb, y_emb, h_emb, w_emb, g_row, b_row, prev)
    return prev


# speedup vs baseline: 1.4013x; 1.0409x over previous
import functools

import jax
import jax.numpy as jnp
from jax import lax
from jax.experimental import pallas as pl
from jax.experimental.pallas import tpu as pltpu
from jax.experimental.pallas import tpu_sc as plsc

B, L, H = 16, 512, 768
PAD = 1
NTOK = B * L
NC, NS = 2, 16
NW = NC * NS
CH = 64
POS_K = 514
SPAT_K = 1024
NCHUNK = 2
BC = B // NCHUNK
NTOKC = BC * L
ROWS_PER_TILE = NTOKC // NW
CHUNKS = ROWS_PER_TILE // CH


def _sc_gather_words(word_emb, idx_flat):
    mesh = plsc.VectorSubcoreMesh(core_axis_name="c", subcore_axis_name="s")

    @functools.partial(
        pl.kernel,
        out_type=jax.ShapeDtypeStruct((NTOKC, H), jnp.float32),
        mesh=mesh,
        scratch_types=[
            pltpu.VMEM((ROWS_PER_TILE,), jnp.int32),
            pltpu.VMEM((CH, H), jnp.float32),
            pltpu.VMEM((CH, H), jnp.float32),
            pltpu.SemaphoreType.DMA,
            pltpu.SemaphoreType.DMA,
        ],
    )
    def gather_kernel(table_hbm, idx_hbm, out_hbm, idx_v, rows0, rows1, sem0, sem1):
        wid = lax.axis_index("s") * NC + lax.axis_index("c")
        base = wid * ROWS_PER_TILE
        pltpu.sync_copy(idx_hbm.at[pl.ds(base, ROWS_PER_TILE)], idx_v)
        bufs = (rows0, rows1)
        sems = (sem0, sem1)

        def start(c):
            return pltpu.async_copy(
                table_hbm.at[idx_v.at[pl.ds(c * CH, CH)]], bufs[c % 2],
                sems[c % 2])

        copies = [start(0)] + ([start(1)] if CHUNKS > 1 else [])
        for c in range(CHUNKS):
            copies[c % 2].wait()
            pltpu.sync_copy(bufs[c % 2], out_hbm.at[pl.ds(base + c * CH, CH)])
            if c + 2 < CHUNKS:
                copies[c % 2] = start(c + 2)

    return gather_kernel(word_emb, idx_flat)


def _tc_body(w_ref, idx_ref, pos_ref, x_ref, y_ref, h_ref, ww_ref,
             g_ref, b_ref, *prev_and_out):
    o_ref = prev_and_out[-1]
    idx = idx_ref[0]

    def onehot_t(row, k):
        io = lax.broadcasted_iota(jnp.int32, (k, L), 0)
        return (io == idx[row:row + 1, :]).astype(jnp.bfloat16)

    def mm_t(oh_t, table_bf):
        return lax.dot_general(oh_t, table_bf, (((0,), (0,)), ((), ())),
                               preferred_element_type=jnp.float32)

    pos_part = mm_t(onehot_t(0, POS_K), pos_ref[...].astype(jnp.bfloat16))
    x_bf = x_ref[...].astype(jnp.bfloat16)
    y_bf = y_ref[...].astype(jnp.bfloat16)
    left = mm_t(onehot_t(1, SPAT_K), x_bf)
    upper = mm_t(onehot_t(2, SPAT_K), y_bf)
    right = mm_t(onehot_t(3, SPAT_K), x_bf)
    lower = mm_t(onehot_t(4, SPAT_K), y_bf)
    hgt = mm_t(onehot_t(5, SPAT_K), h_ref[...].astype(jnp.bfloat16))
    wid = mm_t(onehot_t(6, SPAT_K), ww_ref[...].astype(jnp.bfloat16))
    spatial = jnp.concatenate([left, upper, right, lower, hgt, wid], axis=-1)

    acc = w_ref[0] + pos_part + spatial
    mu = jnp.mean(acc, axis=-1, keepdims=True)
    d = acc - mu
    var = jnp.mean(d * d, axis=-1, keepdims=True)
    o_ref[0] = d * lax.rsqrt(var + 1e-5) * g_ref[...] + b_ref[...]


def _tc_fuse(chunk, w_rows, idx_t, pos_t, x_emb, y_emb, h_emb, w_emb,
             g_row, b_row, prev):
    base = chunk * BC
    specs = [
        pl.BlockSpec((1, L, H), lambda i: (i, 0, 0)),
        pl.BlockSpec((1, 8, L), lambda i: (i, 0, 0)),
        pl.BlockSpec((POS_K, H), lambda i: (0, 0)),
        pl.BlockSpec((SPAT_K, 128), lambda i: (0, 0)),
        pl.BlockSpec((SPAT_K, 128), lambda i: (0, 0)),
        pl.BlockSpec((SPAT_K, 128), lambda i: (0, 0)),
        pl.BlockSpec((SPAT_K, 128), lambda i: (0, 0)),
        pl.BlockSpec((1, H), lambda i: (0, 0)),
        pl.BlockSpec((1, H), lambda i: (0, 0)),
    ]
    args = [w_rows, idx_t, pos_t, x_emb, y_emb, h_emb, w_emb,
            g_row, b_row]
    aliases = {}
    if prev is not None:
        specs.append(pl.BlockSpec(memory_space=pl.ANY))
        args.append(prev)
        aliases = {9: 0}
    return pl.pallas_call(
        _tc_body,
        grid=(BC,),
        compiler_params=pltpu.CompilerParams(
            dimension_semantics=("arbitrary",)),
        in_specs=specs,
        out_specs=pl.BlockSpec((1, L, H), lambda i, b=base: (b + i, 0, 0)),
        out_shape=jax.ShapeDtypeStruct((B, L, H), jnp.float32),
        input_output_aliases=aliases,
    )(*args)


def kernel(input_ids, bbox, word_emb, token_type_emb, pos_emb, x_emb, y_emb,
           h_emb, w_emb, ln_g, ln_b):
    mask = (input_ids != PAD).astype(jnp.int32)
    pids = jnp.cumsum(mask, axis=1) * mask + PAD
    b0 = bbox[:, :, 0]
    b1 = bbox[:, :, 1]
    b2 = bbox[:, :, 2]
    b3 = bbox[:, :, 3]
    hi = jnp.clip(b3 - b1, 0, SPAT_K - 1)
    wi = jnp.clip(b2 - b0, 0, SPAT_K - 1)
    idx_t = jnp.stack([pids, b0, b1, b2, b3, hi, wi, pids], axis=1)

    pos_t = pos_emb + token_type_emb[0:1]
    g_row = ln_g.reshape(1, H)
    b_row = ln_b.reshape(1, H)

    idx_flat = input_ids.reshape(NCHUNK, NTOKC)
    idx_t_c = idx_t.reshape(NCHUNK, BC, 8, L)

    w_chunks = [_sc_gather_words(word_emb, idx_flat[k]).reshape(BC, L, H)
                for k in range(NCHUNK)]
    prev = None
    for k in range(NCHUNK):
        prev = _tc_fuse(k, w_chunks[k], idx_t_c[k], pos_t,
                        x_emb, y_emb, h_emb, w_emb, g_row, b_row, prev)
    return prev
